# Initial kernel scaffold; baseline (speedup 1.0000x reference)
import jax
import jax.numpy as jnp
from jax.experimental import pallas as pl


def kernel(x, P, imbalance):
    raise NotImplementedError("write your pallas kernel here")


# speedup vs baseline: 2.1554x; 2.1554x over previous
import functools

import jax
import jax.numpy as jnp
from jax import lax
from jax.experimental import pallas as pl
from jax.experimental.pallas import tpu as pltpu
from jax.experimental.pallas import tpu_sc as plsc

N = 100000
D = 128
NC = 2
NS = 16
NW = NC * NS
CHUNK = 3136
NIT = CHUNK // 16
BETA = 1.0


def _body(x_hbm, p_hbm, im_hbm, part_hbm, psrc_v, p_v, im_v, out_v, sem):
    c = lax.axis_index("c")
    s = lax.axis_index("s")
    wid = s * NC + c
    start = wid * CHUNK
    base = jnp.minimum(start, N - CHUNK)

    cp0 = pltpu.async_copy(x_hbm.at[pl.ds(base, CHUNK), pl.ds(D - 16, 16)], psrc_v, sem)
    cp1 = pltpu.async_copy(p_hbm.at[pl.ds(base, CHUNK)], p_v, sem)
    cp2 = pltpu.async_copy(im_hbm.at[pl.ds(base, CHUNK)], im_v, sem)
    cp0.wait()
    cp1.wait()
    cp2.wait()

    iota = lax.iota(jnp.int32, 16)
    lane15 = jnp.full((16,), 15, jnp.int32)
    zero_f = jnp.zeros((16,), jnp.float32)
    one_f = jnp.ones((16,), jnp.float32)

    def step(i, carry):
        a_dp, a_np, a_iz, a_nz = carry
        off = i * 16
        idx = off + iota
        ps = plsc.load_gather(psrc_v, [idx, lane15])
        p = p_v[pl.ds(off, 16)]
        im = im_v[pl.ds(off, 16)]
        valid = (base + idx) >= start
        mpos = jnp.where(jnp.logical_and(ps > 0, valid), one_f, zero_f)
        mzero = jnp.where(jnp.logical_and(ps == 0, valid), one_f, zero_f)
        d = ps - p
        return (
            a_dp + d * d * mpos,
            a_np + mpos,
            a_iz + im * im * mzero,
            a_nz + mzero,
        )

    acc = (zero_f, zero_f, zero_f, zero_f)
    a_dp, a_np, a_iz, a_nz = lax.fori_loop(0, NIT, step, acc)

    s_dp = jnp.sum(a_dp)
    s_np = jnp.sum(a_np)
    s_iz = jnp.sum(a_iz)
    s_nz = jnp.sum(a_nz)

    res = (
        jnp.where(iota == 0, s_dp, 0.0)
        + jnp.where(iota == 1, s_np, 0.0)
        + jnp.where(iota == 2, s_iz, 0.0)
        + jnp.where(iota == 3, s_nz, 0.0)
    )
    out_v[...] = res
    pltpu.sync_copy(out_v, part_hbm.at[wid])


@jax.jit
def _partials(x, p_flat, imbalance):
    mesh = plsc.VectorSubcoreMesh(
        core_axis_name="c", subcore_axis_name="s", num_cores=NC, num_subcores=NS
    )
    return pl.kernel(
        _body,
        out_type=jax.ShapeDtypeStruct((NW, 16), jnp.float32),
        mesh=mesh,
        scratch_types=[
            pltpu.VMEM((CHUNK, 16), jnp.float32),
            pltpu.VMEM((CHUNK,), jnp.float32),
            pltpu.VMEM((CHUNK,), jnp.float32),
            pltpu.VMEM((16,), jnp.float32),
            pltpu.SemaphoreType.DMA,
        ],
        compiler_params=pltpu.CompilerParams(
            use_tc_tiling_on_sc=False, needs_layout_passes=False
        ),
    )(x, p_flat, imbalance)


def kernel(x, P, imbalance):
    part = _partials(x, P.reshape(-1), imbalance)
    sums = jnp.sum(part, axis=0)
    pl_ = sums[0] / sums[1]
    ql = sums[2] / sums[3]
    return BETA * ql + (1.0 - BETA) * pl_
